# Initial kernel scaffold; baseline (speedup 1.0000x reference)
import jax
import jax.numpy as jnp
from jax.experimental import pallas as pl


def kernel(x, edge_index, batch, W1, b1, W2, b2, W3, b3, W4, b4, mu_W, mu_b, lv_W, lv_b, dec1_W, dec1_b, adj_W, adj_b, node_W, node_b):
    raise NotImplementedError("write your pallas kernel here")


# speedup vs baseline: 2.7554x; 2.7554x over previous
import jax
import jax.numpy as jnp
from jax import lax
from jax.experimental import pallas as pl
from jax.experimental.pallas import tpu as pltpu
from jax.experimental.pallas import tpu_sc as plsc

N = 10000
E = 320000
D = 128
H = 128
LDIM = 64
G = 64
MAXN = 160

NC = 2
NS = 16
E2 = E // NC
CH = 1280
NCHUNK = E2 // CH
HEPT = E // (NC * NS)

NB = 2000
GRID = N // NB


def _mesh():
    return plsc.VectorSubcoreMesh(
        core_axis_name="c", subcore_axis_name="s", num_cores=NC, num_subcores=NS
    )


def _sc_hist_body(dst_hbm, hist_hbm, tbl, dbuf):
    c = lax.axis_index("c")
    s = lax.axis_index("s")
    wid = c * NS + s
    lane = lax.iota(jnp.int32, 16)
    one = jnp.ones((16,), jnp.float32)
    zv = jnp.zeros((16,), jnp.float32)

    def zloop(i, _):
        tbl[i, :] = zv
        return 0

    lax.fori_loop(0, N // 16, zloop, 0, unroll=4)
    pltpu.sync_copy(dst_hbm.at[pl.ds(wid * HEPT, HEPT)], dbuf)

    def grp(jj, _):
        dv = dbuf[pl.ds(jj * 16, 16)]
        for k in range(16):
            plsc.addupdate_scatter(tbl, [dv >> 4, dv & 15], one, mask=lane == k)
        return 0

    lax.fori_loop(0, HEPT // 16, grp, 0)
    pltpu.sync_copy(tbl, hist_hbm.at[wid])


_hist = pl.kernel(
    _sc_hist_body,
    out_type=jax.ShapeDtypeStruct((NC * NS, N // 16, 16), jnp.float32),
    mesh=_mesh(),
    compiler_params=pltpu.CompilerParams(needs_layout_passes=False, use_tc_tiling_on_sc=False),
    scratch_types=[
        pltpu.VMEM((N // 16, 16), jnp.float32),
        pltpu.VMEM((HEPT,), jnp.int32),
    ],
)


def _sc_scatter_body(g_hbm, src_hbm, dst_hbm, out_hbm, tbl, srcb, dstb, idxb,
                     rows, semg):
    c = lax.axis_index("c")
    s = lax.axis_index("s")
    lane = lax.iota(jnp.int32, 16)
    lane07 = lane & 7
    half = lane >> 3
    masklo = lane < 8
    maskhi = lane >= 8
    zv = jnp.zeros((16,), jnp.float32)

    def zloop(i, _):
        plsc.store_scatter(tbl, [2 * i + half, lane07], zv)
        return 0

    lax.fori_loop(0, N // 2, zloop, 0, unroll=4)

    ebase = c * E2

    def chunk(bk, _):
        e0 = ebase + bk * CH
        pltpu.sync_copy(src_hbm.at[pl.ds(e0, CH)], srcb)
        pltpu.sync_copy(dst_hbm.at[pl.ds(e0, CH)], dstb)

        def mkidx(k, __):
            idxb[pl.ds(k * 16, 16)] = srcb[pl.ds(k * 16, 16)] * NS + s
            return 0

        lax.fori_loop(0, CH // 16, mkidx, 0, unroll=4)

        cps = [
            pltpu.async_copy(
                g_hbm.at[idxb.at[pl.ds(r * 128, 128)]],
                rows.at[pl.ds(r * 128, 128)],
                semg,
            )
            for r in range(CH // 128)
        ]
        for cp in cps:
            cp.wait()

        def grp(jj, __):
            eb = jj * 16
            for p in range(8):
                eidx = eb + 2 * p + half
                vals = plsc.load_gather(rows, [eidx, lane07])
                dv = plsc.load_gather(dstb, [eidx])
                plsc.addupdate_scatter(tbl, [dv, lane07], vals, mask=masklo)
                plsc.addupdate_scatter(tbl, [dv, lane07], vals, mask=maskhi)
            return 0

        lax.fori_loop(0, CH // 16, grp, 0)
        return 0

    lax.fori_loop(0, NCHUNK, chunk, 0)
    pltpu.sync_copy(tbl, out_hbm.at[c, :, s, :])


_scatter = pl.kernel(
    _sc_scatter_body,
    out_type=jax.ShapeDtypeStruct((NC, N, NS, 8), jnp.float32),
    mesh=_mesh(),
    compiler_params=pltpu.CompilerParams(needs_layout_passes=False, use_tc_tiling_on_sc=False),
    scratch_types=[
        pltpu.VMEM((N, 8), jnp.float32),
        pltpu.VMEM((CH,), jnp.int32),
        pltpu.VMEM((CH,), jnp.int32),
        pltpu.VMEM((CH,), jnp.int32),
        pltpu.VMEM((CH, 8), jnp.float32),
        pltpu.SemaphoreType.DMA,
    ],
)


def _tc_dis_body(hist_ref, dis_ref):
    ones = jnp.ones((NC * NS, 1), jnp.float32)
    deg = lax.dot_general(hist_ref[...], ones, (((0,), (0,)), ((), ())),
                          preferred_element_type=jnp.float32) + 1.0
    dis_ref[...] = lax.rsqrt(deg)


def _tc_dis(hist):
    return pl.pallas_call(
        _tc_dis_body,
        out_shape=jax.ShapeDtypeStruct((N, 1), jnp.float32),
    )(hist)


def _tc_first_body(dis_ref, x_ref, w1_ref, g_ref):
    t = jnp.dot(x_ref[...], w1_ref[...], preferred_element_type=jnp.float32)
    g_ref[...] = t * dis_ref[...]


def _tc_first(dis, x, W1):
    return pl.pallas_call(
        _tc_first_body,
        grid=(GRID,),
        in_specs=[
            pl.BlockSpec((NB, 1), lambda i: (i, 0)),
            pl.BlockSpec((NB, D), lambda i: (i, 0)),
            pl.BlockSpec((D, H), lambda i: (0, 0)),
        ],
        out_specs=pl.BlockSpec((NB, H), lambda i: (i, 0)),
        out_shape=jax.ShapeDtypeStruct((N, H), jnp.float32),
    )(dis, x, W1)


def _tc_mid_body(pp_ref, gprev_ref, dis_ref, b_ref, w_ref, gout_ref):
    a = pp_ref[...]
    dis = dis_ref[...]
    h = jnp.maximum((a[0] + a[1] + gprev_ref[...]) * dis + b_ref[...], 0.0)
    t = jnp.dot(h, w_ref[...], preferred_element_type=jnp.float32)
    gout_ref[...] = t * dis


def _tc_mid(pp, gprev, dis, b, W):
    return pl.pallas_call(
        _tc_mid_body,
        grid=(GRID,),
        in_specs=[
            pl.BlockSpec((NC, NB, H), lambda i: (0, i, 0)),
            pl.BlockSpec((NB, H), lambda i: (i, 0)),
            pl.BlockSpec((NB, 1), lambda i: (i, 0)),
            pl.BlockSpec((1, H), lambda i: (0, 0)),
            pl.BlockSpec((H, H), lambda i: (0, 0)),
        ],
        out_specs=pl.BlockSpec((NB, H), lambda i: (i, 0)),
        out_shape=jax.ShapeDtypeStruct((N, H), jnp.float32),
    )(pp, gprev, dis, b, W)


def _tc_enc_body(pp_ref, g4_ref, dis_ref, b4_ref, batch_ref, muW_ref, mub_ref,
                 lvW_ref, lvb_ref, d1W_ref, d1b_ref, eps_ref, mu_ref, lv_ref,
                 xd_ref, acc_ref):
    i = pl.program_id(0)
    a = pp_ref[...]
    dis = dis_ref[...]
    h = jnp.maximum((a[0] + a[1] + g4_ref[...]) * dis + b4_ref[...], 0.0)
    gid = lax.broadcasted_iota(jnp.int32, (NB, G), 1).astype(jnp.float32)
    onehot = (gid == batch_ref[...]).astype(jnp.float32)
    seg = lax.dot_general(onehot, h, (((0,), (0,)), ((), ())),
                          preferred_element_type=jnp.float32)

    @pl.when(i == 0)
    def _():
        acc_ref[...] = seg

    @pl.when(i > 0)
    def _():
        acc_ref[...] = acc_ref[...] + seg

    @pl.when(i == GRID - 1)
    def _():
        pooled = acc_ref[...]
        mu = jnp.dot(pooled, muW_ref[...],
                     preferred_element_type=jnp.float32) + mub_ref[...]
        lv = jnp.dot(pooled, lvW_ref[...],
                     preferred_element_type=jnp.float32) + lvb_ref[...]
        std = jnp.exp(0.5 * lv)
        z = mu + eps_ref[...] * std
        xd = jnp.maximum(
            jnp.dot(z, d1W_ref[...], preferred_element_type=jnp.float32)
            + d1b_ref[...], 0.0)
        mu_ref[...] = mu
        lv_ref[...] = lv
        xd_ref[...] = xd


def _tc_enc(pp, g4, dis, b4, batchf, muW, mub, lvW, lvb, d1W, d1b, eps):
    full = lambda shape: pl.BlockSpec(shape, lambda i: tuple(0 for _ in shape))
    return pl.pallas_call(
        _tc_enc_body,
        grid=(GRID,),
        in_specs=[
            pl.BlockSpec((NC, NB, H), lambda i: (0, i, 0)),
            pl.BlockSpec((NB, H), lambda i: (i, 0)),
            pl.BlockSpec((NB, 1), lambda i: (i, 0)),
            full((1, H)),
            pl.BlockSpec((NB, 1), lambda i: (i, 0)),
            full((H, LDIM)),
            full((1, LDIM)),
            full((H, LDIM)),
            full((1, LDIM)),
            full((LDIM, 256)),
            full((1, 256)),
            full((G, LDIM)),
        ],
        out_specs=[full((G, LDIM)), full((G, LDIM)), full((G, 256))],
        out_shape=[
            jax.ShapeDtypeStruct((G, LDIM), jnp.float32),
            jax.ShapeDtypeStruct((G, LDIM), jnp.float32),
            jax.ShapeDtypeStruct((G, 256), jnp.float32),
        ],
        scratch_shapes=[pltpu.VMEM((G, H), jnp.float32)],
    )(pp, g4, dis, b4, batchf, muW, mub, lvW, lvb, d1W, d1b, eps)


ABLK = MAXN * MAXN // 8
NBLK = MAXN * D // 8


def _tc_dec_body(xd_ref, aW_ref, ab_ref, nW_ref, nb_ref, adj_ref, node_ref):
    xd = xd_ref[...]
    adj_ref[...] = jax.nn.sigmoid(
        jnp.dot(xd, aW_ref[...], preferred_element_type=jnp.float32)
        + ab_ref[...])
    node_ref[...] = jnp.dot(
        xd, nW_ref[...], preferred_element_type=jnp.float32) + nb_ref[...]


def _tc_dec(xd, aW, ab, nW, nb):
    return pl.pallas_call(
        _tc_dec_body,
        grid=(8,),
        in_specs=[
            pl.BlockSpec((G, 256), lambda i: (0, 0)),
            pl.BlockSpec((256, ABLK), lambda i: (0, i)),
            pl.BlockSpec((1, ABLK), lambda i: (0, i)),
            pl.BlockSpec((256, NBLK), lambda i: (0, i)),
            pl.BlockSpec((1, NBLK), lambda i: (0, i)),
        ],
        out_specs=[
            pl.BlockSpec((G, ABLK), lambda i: (0, i)),
            pl.BlockSpec((G, NBLK), lambda i: (0, i)),
        ],
        out_shape=[
            jax.ShapeDtypeStruct((G, MAXN * MAXN), jnp.float32),
            jax.ShapeDtypeStruct((G, MAXN * D), jnp.float32),
        ],
    )(xd, aW, ab, nW, nb)


def kernel(x, edge_index, batch, W1, b1, W2, b2, W3, b3, W4, b4, mu_W, mu_b,
           lv_W, lv_b, dec1_W, dec1_b, adj_W, adj_b, node_W, node_b):
    src = edge_index[0]
    dst = edge_index[1]

    hist = _hist(dst).reshape(NC * NS, N)
    dis = _tc_dis(hist)
    g1 = _tc_first(dis, x, W1)

    p = _scatter(g1.reshape(N * NS, 8), src, dst).reshape(NC, N, H)
    g2 = _tc_mid(p, g1, dis, b1.reshape(1, H), W2)
    p = _scatter(g2.reshape(N * NS, 8), src, dst).reshape(NC, N, H)
    g3 = _tc_mid(p, g2, dis, b2.reshape(1, H), W3)
    p = _scatter(g3.reshape(N * NS, 8), src, dst).reshape(NC, N, H)
    g4 = _tc_mid(p, g3, dis, b3.reshape(1, H), W4)
    p = _scatter(g4.reshape(N * NS, 8), src, dst).reshape(NC, N, H)

    eps = jax.random.normal(jax.random.key(42), (G, LDIM), jnp.float32)
    batchf = batch.astype(jnp.float32).reshape(N, 1)
    mu, logvar, xd = _tc_enc(p, g4, dis, b4.reshape(1, H), batchf, mu_W,
                             mu_b.reshape(1, LDIM), lv_W, lv_b.reshape(1, LDIM),
                             dec1_W, dec1_b.reshape(1, 256), eps)
    adj, node = _tc_dec(xd, adj_W, adj_b.reshape(1, MAXN * MAXN), node_W,
                        node_b.reshape(1, MAXN * D))
    return (adj.reshape(G, MAXN, MAXN), node.reshape(G, MAXN, D), mu, logvar)
